# baseline (speedup 1.0000x reference)
import functools
import jax
import jax.numpy as jnp
from jax import lax
from jax.experimental import pallas as pl
from jax.experimental.pallas import tpu as pltpu
from jax.experimental.pallas import tpu_sc as plsc


_ROWS = 16384
_K = 128
_NC = 2
_NS = 16
_NW = _NC * _NS
_ROWS_PER_W = _ROWS // _NW
_CHUNK_ROWS = 64
_NCHUNK = _ROWS_PER_W // _CHUNK_ROWS
_FIRE = 16


def _sc_body(x_hbm, idx_hbm, out_hbm, idx_v, data_v, sem):
    wid = lax.axis_index("s") * _NC + lax.axis_index("c")
    row0 = wid * _ROWS_PER_W

    def chunk(ci, _):
        base = row0 + ci * _CHUNK_ROWS
        pltpu.sync_copy(idx_hbm.at[pl.ds(base, _CHUNK_ROWS)], idx_v)
        for g in range(0, _CHUNK_ROWS, _FIRE):
            handles = [
                pltpu.async_copy(x_hbm.at[idx_v.at[j]], data_v.at[j], sem)
                for j in range(g, g + _FIRE)
            ]
            for h in handles:
                h.wait()
        pltpu.sync_copy(data_v, out_hbm.at[pl.ds(base, _CHUNK_ROWS)])
        return ()

    lax.fori_loop(0, _NCHUNK, chunk, ())


def kernel(x, w):
    b, srows, cols = x.shape
    k = w.shape[0]
    xf = x.reshape(b * srows * cols)
    idx = (
        jnp.arange(b * srows, dtype=jnp.int32)[:, None] * cols + w[None, :]
    )

    mesh = plsc.VectorSubcoreMesh(core_axis_name="c", subcore_axis_name="s")
    f = functools.partial(
        pl.kernel,
        mesh=mesh,
        out_type=jax.ShapeDtypeStruct((b * srows, k), jnp.float32),
        scratch_types=[
            pltpu.VMEM((_CHUNK_ROWS, _K), jnp.int32),
            pltpu.VMEM((_CHUNK_ROWS, _K), jnp.float32),
            pltpu.SemaphoreType.DMA,
        ],
    )(_sc_body)
    out = f(xf, idx)
    return out.reshape(b, srows, k)


# speedup vs baseline: 1.0496x; 1.0496x over previous
import functools
import jax
import jax.numpy as jnp
from jax import lax
from jax.experimental import pallas as pl
from jax.experimental.pallas import tpu as pltpu
from jax.experimental.pallas import tpu_sc as plsc


_ROWS = 16384
_K = 128
_NC = 2
_NS = 16
_NW = _NC * _NS
_ROWS_PER_W = _ROWS // _NW
_CHUNK_ROWS = 64
_NCHUNK = _ROWS_PER_W // _CHUNK_ROWS
_FIRE = 16


def _sc_body(x_hbm, idx_hbm, out_hbm, idx_v, data_v, sem):
    wid = lax.axis_index("s") * _NC + lax.axis_index("c")
    row0 = wid * _ROWS_PER_W

    def chunk(ci, _):
        base = (row0 + ci * _CHUNK_ROWS) * _K
        pltpu.sync_copy(idx_hbm.at[pl.ds(base, _CHUNK_ROWS * _K)], idx_v)
        pltpu.async_copy(x_hbm.at[idx_v], data_v, sem).wait()
        pltpu.sync_copy(data_v, out_hbm.at[pl.ds(base, _CHUNK_ROWS * _K)])
        return ()

    lax.fori_loop(0, _NCHUNK, chunk, ())


def kernel(x, w):
    b, srows, cols = x.shape
    k = w.shape[0]
    xf = x.reshape(b * srows * cols)
    idx = (
        jnp.arange(b * srows, dtype=jnp.int32)[:, None] * cols + w[None, :]
    ).reshape(b * srows * k)

    mesh = plsc.VectorSubcoreMesh(core_axis_name="c", subcore_axis_name="s")
    f = functools.partial(
        pl.kernel,
        mesh=mesh,
        out_type=jax.ShapeDtypeStruct((b * srows * k,), jnp.float32),
        scratch_types=[
            pltpu.VMEM((_CHUNK_ROWS * _K,), jnp.int32),
            pltpu.VMEM((_CHUNK_ROWS * _K,), jnp.float32),
            pltpu.SemaphoreType.DMA,
        ],
    )(_sc_body)
    out = f(xf, idx)
    return out.reshape(b, srows, k)


# speedup vs baseline: 3.5220x; 3.3556x over previous
import jax
import jax.numpy as jnp
from jax.experimental import pallas as pl


_BLOCK_R = 1024


def _select_body(x_ref, s_ref, o_ref):
    o_ref[...] = jnp.dot(
        x_ref[...], s_ref[...], preferred_element_type=jnp.float32
    )


def kernel(x, w):
    b, srows, cols = x.shape
    k = w.shape[0]
    xf = x.reshape(b * srows, cols)
    sel = (
        jax.lax.broadcasted_iota(jnp.int32, (cols, k), 0) == w[None, :]
    ).astype(jnp.float32)

    grid = (xf.shape[0] // _BLOCK_R,)
    out = pl.pallas_call(
        _select_body,
        grid=grid,
        in_specs=[
            pl.BlockSpec((_BLOCK_R, cols), lambda i: (i, 0)),
            pl.BlockSpec((cols, k), lambda i: (0, 0)),
        ],
        out_specs=pl.BlockSpec((_BLOCK_R, k), lambda i: (i, 0)),
        out_shape=jax.ShapeDtypeStruct((xf.shape[0], k), jnp.float32),
    )(xf, sel)
    return out.reshape(b, srows, k)
